# Initial kernel scaffold; baseline (speedup 1.0000x reference)
import jax
import jax.numpy as jnp
from jax.experimental import pallas as pl


def kernel(memory, last_update, n_id, dst_s, src_d, t_s, t_d, raw_msg_s, raw_msg_d, time_w, time_b, W_ih, W_hh, b_ih, b_hh):
    raise NotImplementedError("write your pallas kernel here")


# speedup vs baseline: 3.3698x; 3.3698x over previous
import functools

import jax
import jax.numpy as jnp
from jax import lax
from jax.experimental import pallas as pl
from jax.experimental.pallas import tpu as pltpu
from jax.experimental.pallas import tpu_sc as plsc

N = 100000
B = 16384
MD = 100
TD = 100
RD = 172
L = 16
NW = 32
BPW = B // NW
RANGE = 3136
NPAD = NW * RANGE

_mesh = plsc.VectorSubcoreMesh(core_axis_name="c", subcore_axis_name="s")


def _wid():
    return lax.axis_index("s") * 2 + lax.axis_index("c")


MP = 128
CH = BPW // 2


@functools.partial(
    pl.kernel,
    mesh=_mesh,
    compiler_params=pltpu.CompilerParams(needs_layout_passes=False),
    out_type=(
        jax.ShapeDtypeStruct((B, MP), jnp.float32),
        jax.ShapeDtypeStruct((B, MP), jnp.float32),
        jax.ShapeDtypeStruct((B,), jnp.float32),
        jax.ShapeDtypeStruct((B,), jnp.float32),
        jax.ShapeDtypeStruct((B,), jnp.int32),
    ),
    scratch_types=[
        pltpu.VMEM((BPW,), jnp.int32),
        pltpu.VMEM((BPW,), jnp.int32),
        pltpu.VMEM((BPW,), jnp.int32),
        pltpu.VMEM((BPW,), jnp.int32),
        pltpu.VMEM((BPW,), jnp.int32),
        pltpu.VMEM((BPW,), jnp.int32),
        pltpu.VMEM((BPW,), jnp.int32),
        pltpu.VMEM((BPW,), jnp.int32),
        pltpu.VMEM((BPW,), jnp.int32),
        pltpu.VMEM((CH, MP), jnp.float32),
        pltpu.VMEM((CH, MP), jnp.float32),
        pltpu.VMEM((BPW,), jnp.int32),
        pltpu.VMEM((BPW,), jnp.float32),
        pltpu.VMEM((BPW,), jnp.float32),
        pltpu.SemaphoreType.DMA,
        pltpu.SemaphoreType.DMA,
        pltpu.SemaphoreType.DMA,
    ],
)
def _sc_gather(mem_hbm, lu_hbm, nid_hbm, dst_hbm, srcd_hbm, ts_hbm, td_hbm,
               h_out, oth_out, trel_out, sel_out, tmax_out,
               nid_v, dst_v, src_v, ts_v, td_v, idx2_v, idxl_v, tev_v, tmx_v,
               hrow_v, orow_v, lu_v, trel_v, sel_v, sem1, sem2, sem3):
    base = _wid() * BPW
    pltpu.sync_copy(nid_hbm.at[pl.ds(base, BPW)], nid_v)
    pltpu.sync_copy(dst_hbm.at[pl.ds(base, BPW)], dst_v)
    pltpu.sync_copy(srcd_hbm.at[pl.ds(base, BPW)], src_v)
    pltpu.sync_copy(ts_hbm.at[pl.ds(base, BPW)], ts_v)
    pltpu.sync_copy(td_hbm.at[pl.ds(base, BPW)], td_v)

    def step(i, carry):
        s = pl.ds(i * L, L)
        ts = ts_v[s]
        td = td_v[s]
        sel = td >= ts
        idx2_v[s] = jnp.where(sel, src_v[s], dst_v[s])
        idxl_v[s] = jnp.where(sel, src_v[s], nid_v[s])
        tev_v[s] = jnp.where(sel, td, ts)
        tmx_v[s] = jnp.maximum(ts, td)
        sel_v[s] = jnp.where(sel, 1.0, 0.0).astype(jnp.float32)
        return carry

    lax.fori_loop(0, BPW // L, step, 0)

    cp3 = pltpu.async_copy(lu_hbm.at[idxl_v], lu_v, sem3)
    for c in range(BPW // CH):
        cp1 = pltpu.async_copy(mem_hbm.at[nid_v.at[pl.ds(c * CH, CH)]], hrow_v, sem1)
        cp2 = pltpu.async_copy(mem_hbm.at[idx2_v.at[pl.ds(c * CH, CH)]], orow_v, sem2)
        cp1.wait()
        cp2.wait()
        pltpu.sync_copy(hrow_v, h_out.at[pl.ds(base + c * CH, CH)])
        pltpu.sync_copy(orow_v, oth_out.at[pl.ds(base + c * CH, CH)])
    cp3.wait()

    def step2(i, carry):
        s = pl.ds(i * L, L)
        trel_v[s] = (tev_v[s] - lu_v[s]).astype(jnp.float32)
        return carry

    lax.fori_loop(0, BPW // L, step2, 0)

    pltpu.sync_copy(trel_v, trel_out.at[pl.ds(base, BPW)])
    pltpu.sync_copy(sel_v, sel_out.at[pl.ds(base, BPW)])
    pltpu.sync_copy(tmx_v, tmax_out.at[pl.ds(base, BPW)])


@functools.partial(
    pl.kernel,
    mesh=_mesh,
    compiler_params=pltpu.CompilerParams(needs_layout_passes=False),
    out_type=jax.ShapeDtypeStruct((NPAD,), jnp.int32),
    scratch_types=[
        pltpu.VMEM((RANGE,), jnp.int32),
        pltpu.VMEM((B,), jnp.int32),
        pltpu.VMEM((B,), jnp.int32),
    ],
)
def _sc_scatter_max(lupad_hbm, nid_hbm, tmax_hbm, luout_hbm, tab_v, nid_v, tmx_v):
    lo = _wid() * RANGE
    pltpu.sync_copy(lupad_hbm.at[pl.ds(lo, RANGE)], tab_v)
    pltpu.sync_copy(nid_hbm, nid_v)
    pltpu.sync_copy(tmax_hbm, tmx_v)

    def step(i, carry):
        s = pl.ds(i * L, L)
        local = nid_v[s] - lo
        tv = tmx_v[s]
        m = (local >= 0) & (local < RANGE)
        cur = plsc.load_gather(tab_v, [local], mask=m)
        act = m & (cur < tv)

        def cond(a):
            return jnp.any(a)

        def body(a):
            plsc.store_scatter(tab_v, [local], tv, mask=a)
            cur2 = plsc.load_gather(tab_v, [local], mask=a)
            return a & (cur2 < tv)

        lax.while_loop(cond, body, act)
        return carry

    lax.fori_loop(0, B // L, step, 0)
    pltpu.sync_copy(tab_v, luout_hbm.at[pl.ds(lo, RANGE)])


@functools.partial(
    pl.kernel,
    mesh=_mesh,
    compiler_params=pltpu.CompilerParams(needs_layout_passes=False),
    out_type=jax.ShapeDtypeStruct((B,), jnp.int32),
    scratch_types=[
        pltpu.VMEM((BPW,), jnp.int32),
        pltpu.VMEM((BPW,), jnp.int32),
        pltpu.SemaphoreType.DMA,
    ],
)
def _sc_last_gather(lut_hbm, nid_hbm, out_hbm, nid_v, val_v, sem):
    base = _wid() * BPW
    pltpu.sync_copy(nid_hbm.at[pl.ds(base, BPW)], nid_v)
    pltpu.async_copy(lut_hbm.at[nid_v], val_v, sem).wait()
    pltpu.sync_copy(val_v, out_hbm.at[pl.ds(base, BPW)])


BLK = 512
G = 384


def _gru_body(sel_ref, trel_ref, h_ref, o_ref, rms_ref, rmd_ref,
              tw_ref, tb_ref, w1_ref, w2_ref, w3_ref, w4_ref,
              whh_ref, bih_ref, bhh_ref, out_ref):
    f32 = jnp.float32
    sel = sel_ref[...]
    h = h_ref[:, 0:MD]
    o = o_ref[:, 0:MD]
    inv = 1.0 - sel
    p1 = sel * o + inv * h
    p2 = sel * h + inv * o
    p3 = sel * rmd_ref[...] + inv * rms_ref[...]
    tenc = jnp.cos(trel_ref[...] * tw_ref[...] + tb_ref[...])
    gi = (jnp.dot(p1, w1_ref[...], preferred_element_type=f32)
          + jnp.dot(p2, w2_ref[...], preferred_element_type=f32)
          + jnp.dot(p3, w3_ref[...], preferred_element_type=f32)
          + jnp.dot(tenc, w4_ref[...], preferred_element_type=f32)
          + bih_ref[...])
    gh = jnp.dot(h, whh_ref[...], preferred_element_type=f32) + bhh_ref[...]
    r = jax.nn.sigmoid(gi[:, 0:128] + gh[:, 0:128])
    z = jax.nn.sigmoid(gi[:, 128:256] + gh[:, 128:256])
    n = jnp.tanh(gi[:, 256:G] + r * gh[:, 256:G])
    out_ref[...] = (1.0 - z[:, 0:MD]) * n[:, 0:MD] + z[:, 0:MD] * h


_gru = pl.pallas_call(
    _gru_body,
    grid=(B // BLK,),
    in_specs=[
        pl.BlockSpec((BLK, 1), lambda i: (i, 0)),
        pl.BlockSpec((BLK, 1), lambda i: (i, 0)),
        pl.BlockSpec((BLK, MP), lambda i: (i, 0)),
        pl.BlockSpec((BLK, MP), lambda i: (i, 0)),
        pl.BlockSpec((BLK, RD), lambda i: (i, 0)),
        pl.BlockSpec((BLK, RD), lambda i: (i, 0)),
        pl.BlockSpec((1, TD), lambda i: (0, 0)),
        pl.BlockSpec((1, TD), lambda i: (0, 0)),
        pl.BlockSpec((MD, G), lambda i: (0, 0)),
        pl.BlockSpec((MD, G), lambda i: (0, 0)),
        pl.BlockSpec((RD, G), lambda i: (0, 0)),
        pl.BlockSpec((TD, G), lambda i: (0, 0)),
        pl.BlockSpec((MD, G), lambda i: (0, 0)),
        pl.BlockSpec((1, G), lambda i: (0, 0)),
        pl.BlockSpec((1, G), lambda i: (0, 0)),
    ],
    out_specs=pl.BlockSpec((BLK, MD), lambda i: (i, 0)),
    out_shape=jax.ShapeDtypeStruct((B, MD), jnp.float32),
)


def _pad_gates(w):
    lead = w.shape[:-1]
    return jnp.pad(w.reshape(lead + (3, MD)),
                   [(0, 0)] * len(lead) + [(0, 0), (0, 28)]).reshape(lead + (G,))


def kernel(memory, last_update, n_id, dst_s, src_d, t_s, t_d,
           raw_msg_s, raw_msg_d, time_w, time_b, W_ih, W_hh, b_ih, b_hh):
    mem_p = jnp.pad(memory, ((0, 0), (0, MP - MD)))
    h_rows, oth_rows, trel, sel, tmax = _sc_gather(
        mem_p, last_update, n_id, dst_s, src_d, t_s, t_d)

    lu_pad = jnp.pad(last_update, (0, NPAD - N))
    lu_tab = _sc_scatter_max(lu_pad, n_id, tmax)
    new_last = _sc_last_gather(lu_tab, n_id)

    w_iht = _pad_gates(W_ih.T)
    w1 = w_iht[0:MD]
    w2 = w_iht[MD:2 * MD]
    w3 = w_iht[2 * MD:2 * MD + RD]
    w4 = w_iht[2 * MD + RD:]
    whh = _pad_gates(W_hh.T)
    bih = _pad_gates(b_ih).reshape(1, G)
    bhh = _pad_gates(b_hh).reshape(1, G)

    new_mem = _gru(sel.reshape(B, 1), trel.reshape(B, 1), h_rows, oth_rows,
                   raw_msg_s, raw_msg_d,
                   time_w.reshape(1, TD), time_b.reshape(1, TD),
                   w1, w2, w3, w4, whh, bih, bhh)
    return new_mem, new_last


# speedup vs baseline: 4.6206x; 1.3712x over previous
import functools

import jax
import jax.numpy as jnp
from jax import lax
from jax.experimental import pallas as pl
from jax.experimental.pallas import tpu as pltpu
from jax.experimental.pallas import tpu_sc as plsc

N = 100000
B = 16384
MD = 100
TD = 100
RD = 172
L = 16
NW = 32
BPW = B // NW
RANGE = 3136
NPAD = NW * RANGE

_mesh = plsc.VectorSubcoreMesh(core_axis_name="c", subcore_axis_name="s")


def _wid():
    return lax.axis_index("s") * 2 + lax.axis_index("c")


MP = 128
CH = BPW // 2


@functools.partial(
    pl.kernel,
    mesh=_mesh,
    compiler_params=pltpu.CompilerParams(needs_layout_passes=False),
    out_type=(
        jax.ShapeDtypeStruct((B, MP), jnp.float32),
        jax.ShapeDtypeStruct((B, MP), jnp.float32),
        jax.ShapeDtypeStruct((B,), jnp.float32),
        jax.ShapeDtypeStruct((B,), jnp.float32),
        jax.ShapeDtypeStruct((B,), jnp.int32),
    ),
    scratch_types=[
        pltpu.VMEM((BPW,), jnp.int32),
        pltpu.VMEM((BPW,), jnp.int32),
        pltpu.VMEM((BPW,), jnp.int32),
        pltpu.VMEM((BPW,), jnp.int32),
        pltpu.VMEM((BPW,), jnp.int32),
        pltpu.VMEM((BPW,), jnp.int32),
        pltpu.VMEM((BPW,), jnp.int32),
        pltpu.VMEM((BPW,), jnp.int32),
        pltpu.VMEM((BPW,), jnp.int32),
        pltpu.VMEM((CH, MP), jnp.float32),
        pltpu.VMEM((CH, MP), jnp.float32),
        pltpu.VMEM((BPW,), jnp.int32),
        pltpu.VMEM((BPW,), jnp.float32),
        pltpu.VMEM((BPW,), jnp.float32),
        pltpu.SemaphoreType.DMA,
        pltpu.SemaphoreType.DMA,
        pltpu.SemaphoreType.DMA,
    ],
)
def _sc_gather(mem_hbm, lu_hbm, nid_hbm, dst_hbm, srcd_hbm, ts_hbm, td_hbm,
               h_out, oth_out, trel_out, sel_out, tmax_out,
               nid_v, dst_v, src_v, ts_v, td_v, idx2_v, idxl_v, tev_v, tmx_v,
               hrow_v, orow_v, lu_v, trel_v, sel_v, sem1, sem2, sem3):
    base = _wid() * BPW
    pltpu.sync_copy(nid_hbm.at[pl.ds(base, BPW)], nid_v)
    pltpu.sync_copy(dst_hbm.at[pl.ds(base, BPW)], dst_v)
    pltpu.sync_copy(srcd_hbm.at[pl.ds(base, BPW)], src_v)
    pltpu.sync_copy(ts_hbm.at[pl.ds(base, BPW)], ts_v)
    pltpu.sync_copy(td_hbm.at[pl.ds(base, BPW)], td_v)

    def step(i, carry):
        s = pl.ds(i * L, L)
        ts = ts_v[s]
        td = td_v[s]
        sel = td >= ts
        idx2_v[s] = jnp.where(sel, src_v[s], dst_v[s])
        idxl_v[s] = jnp.where(sel, src_v[s], nid_v[s])
        tev_v[s] = jnp.where(sel, td, ts)
        tmx_v[s] = jnp.maximum(ts, td)
        sel_v[s] = jnp.where(sel, 1.0, 0.0).astype(jnp.float32)
        return carry

    lax.fori_loop(0, BPW // L, step, 0)

    cp3 = pltpu.async_copy(lu_hbm.at[idxl_v], lu_v, sem3)
    for c in range(BPW // CH):
        cp1 = pltpu.async_copy(mem_hbm.at[nid_v.at[pl.ds(c * CH, CH)]], hrow_v, sem1)
        cp2 = pltpu.async_copy(mem_hbm.at[idx2_v.at[pl.ds(c * CH, CH)]], orow_v, sem2)
        cp1.wait()
        cp2.wait()
        pltpu.sync_copy(hrow_v, h_out.at[pl.ds(base + c * CH, CH)])
        pltpu.sync_copy(orow_v, oth_out.at[pl.ds(base + c * CH, CH)])
    cp3.wait()

    def step2(i, carry):
        s = pl.ds(i * L, L)
        trel_v[s] = (tev_v[s] - lu_v[s]).astype(jnp.float32)
        return carry

    lax.fori_loop(0, BPW // L, step2, 0)

    pltpu.sync_copy(trel_v, trel_out.at[pl.ds(base, BPW)])
    pltpu.sync_copy(sel_v, sel_out.at[pl.ds(base, BPW)])
    pltpu.sync_copy(tmx_v, tmax_out.at[pl.ds(base, BPW)])


@functools.partial(
    pl.kernel,
    mesh=_mesh,
    compiler_params=pltpu.CompilerParams(needs_layout_passes=False),
    out_type=jax.ShapeDtypeStruct((NPAD,), jnp.int32),
    scratch_types=[
        pltpu.VMEM((RANGE,), jnp.int32),
        pltpu.VMEM((B,), jnp.int32),
        pltpu.VMEM((B,), jnp.int32),
    ],
)
def _sc_scatter_max(lupad_hbm, nid_hbm, tmax_hbm, luout_hbm, tab_v, nid_v, tmx_v):
    lo = _wid() * RANGE
    pltpu.sync_copy(lupad_hbm.at[pl.ds(lo, RANGE)], tab_v)
    pltpu.sync_copy(nid_hbm, nid_v)
    pltpu.sync_copy(tmax_hbm, tmx_v)

    def step(i, carry):
        s = pl.ds(i * L, L)
        local = nid_v[s] - lo
        tv = tmx_v[s]
        m = (local >= 0) & (local < RANGE)
        cur = plsc.load_gather(tab_v, [local], mask=m)
        act = m & (cur < tv)

        def cond(a):
            return jnp.any(a)

        def body(a):
            plsc.store_scatter(tab_v, [local], tv, mask=a)
            cur2 = plsc.load_gather(tab_v, [local], mask=a)
            return a & (cur2 < tv)

        lax.while_loop(cond, body, act)
        return carry

    lax.fori_loop(0, B // L, step, 0)
    pltpu.sync_copy(tab_v, luout_hbm.at[pl.ds(lo, RANGE)])


@functools.partial(
    pl.kernel,
    mesh=_mesh,
    compiler_params=pltpu.CompilerParams(needs_layout_passes=False),
    out_type=jax.ShapeDtypeStruct((B,), jnp.int32),
    scratch_types=[
        pltpu.VMEM((BPW,), jnp.int32),
        pltpu.VMEM((BPW,), jnp.int32),
        pltpu.SemaphoreType.DMA,
    ],
)
def _sc_last_gather(lut_hbm, nid_hbm, out_hbm, nid_v, val_v, sem):
    base = _wid() * BPW
    pltpu.sync_copy(nid_hbm.at[pl.ds(base, BPW)], nid_v)
    pltpu.async_copy(lut_hbm.at[nid_v], val_v, sem).wait()
    pltpu.sync_copy(val_v, out_hbm.at[pl.ds(base, BPW)])


PADBLK = 2000


def _pad_body(in_ref, out_ref):
    out_ref[...] = jnp.concatenate(
        [in_ref[...], jnp.zeros((PADBLK, MP - MD), jnp.float32)], axis=1)


_pad_mem = pl.pallas_call(
    _pad_body,
    grid=(N // PADBLK,),
    in_specs=[pl.BlockSpec((PADBLK, MD), lambda i: (i, 0))],
    out_specs=pl.BlockSpec((PADBLK, MP), lambda i: (i, 0)),
    out_shape=jax.ShapeDtypeStruct((N, MP), jnp.float32),
)


BLK = 512
G = 384


def _gru_body(sel_ref, trel_ref, h_ref, o_ref, rms_ref, rmd_ref,
              tw_ref, tb_ref, w1_ref, w2_ref, w3_ref, w4_ref,
              whh_ref, bih_ref, bhh_ref, out_ref):
    f32 = jnp.float32
    sel = sel_ref[...]
    h = h_ref[:, 0:MD]
    o = o_ref[:, 0:MD]
    inv = 1.0 - sel
    p1 = sel * o + inv * h
    p2 = sel * h + inv * o
    p3 = sel * rmd_ref[...] + inv * rms_ref[...]
    tenc = jnp.cos(trel_ref[...] * tw_ref[...] + tb_ref[...])
    gi = (jnp.dot(p1, w1_ref[...], preferred_element_type=f32)
          + jnp.dot(p2, w2_ref[...], preferred_element_type=f32)
          + jnp.dot(p3, w3_ref[...], preferred_element_type=f32)
          + jnp.dot(tenc, w4_ref[...], preferred_element_type=f32)
          + bih_ref[...])
    gh = jnp.dot(h, whh_ref[...], preferred_element_type=f32) + bhh_ref[...]
    r = jax.nn.sigmoid(gi[:, 0:128] + gh[:, 0:128])
    z = jax.nn.sigmoid(gi[:, 128:256] + gh[:, 128:256])
    n = jnp.tanh(gi[:, 256:G] + r * gh[:, 256:G])
    out_ref[...] = (1.0 - z[:, 0:MD]) * n[:, 0:MD] + z[:, 0:MD] * h


_gru = pl.pallas_call(
    _gru_body,
    grid=(B // BLK,),
    in_specs=[
        pl.BlockSpec((BLK, 1), lambda i: (i, 0)),
        pl.BlockSpec((BLK, 1), lambda i: (i, 0)),
        pl.BlockSpec((BLK, MP), lambda i: (i, 0)),
        pl.BlockSpec((BLK, MP), lambda i: (i, 0)),
        pl.BlockSpec((BLK, RD), lambda i: (i, 0)),
        pl.BlockSpec((BLK, RD), lambda i: (i, 0)),
        pl.BlockSpec((1, TD), lambda i: (0, 0)),
        pl.BlockSpec((1, TD), lambda i: (0, 0)),
        pl.BlockSpec((MD, G), lambda i: (0, 0)),
        pl.BlockSpec((MD, G), lambda i: (0, 0)),
        pl.BlockSpec((RD, G), lambda i: (0, 0)),
        pl.BlockSpec((TD, G), lambda i: (0, 0)),
        pl.BlockSpec((MD, G), lambda i: (0, 0)),
        pl.BlockSpec((1, G), lambda i: (0, 0)),
        pl.BlockSpec((1, G), lambda i: (0, 0)),
    ],
    out_specs=pl.BlockSpec((BLK, MD), lambda i: (i, 0)),
    out_shape=jax.ShapeDtypeStruct((B, MD), jnp.float32),
)


def _pad_gates(w):
    lead = w.shape[:-1]
    return jnp.pad(w.reshape(lead + (3, MD)),
                   [(0, 0)] * len(lead) + [(0, 0), (0, 28)]).reshape(lead + (G,))


def kernel(memory, last_update, n_id, dst_s, src_d, t_s, t_d,
           raw_msg_s, raw_msg_d, time_w, time_b, W_ih, W_hh, b_ih, b_hh):
    mem_p = _pad_mem(memory)
    h_rows, oth_rows, trel, sel, tmax = _sc_gather(
        mem_p, last_update, n_id, dst_s, src_d, t_s, t_d)

    lu_pad = jnp.pad(last_update, (0, NPAD - N))
    lu_tab = _sc_scatter_max(lu_pad, n_id, tmax)
    new_last = _sc_last_gather(lu_tab, n_id)

    w_iht = _pad_gates(W_ih.T)
    w1 = w_iht[0:MD]
    w2 = w_iht[MD:2 * MD]
    w3 = w_iht[2 * MD:2 * MD + RD]
    w4 = w_iht[2 * MD + RD:]
    whh = _pad_gates(W_hh.T)
    bih = _pad_gates(b_ih).reshape(1, G)
    bhh = _pad_gates(b_hh).reshape(1, G)

    new_mem = _gru(sel.reshape(B, 1), trel.reshape(B, 1), h_rows, oth_rows,
                   raw_msg_s, raw_msg_d,
                   time_w.reshape(1, TD), time_b.reshape(1, TD),
                   w1, w2, w3, w4, whh, bih, bhh)
    return new_mem, new_last


# speedup vs baseline: 5.8696x; 1.2703x over previous
import functools

import jax
import jax.numpy as jnp
from jax import lax
from jax.experimental import pallas as pl
from jax.experimental.pallas import tpu as pltpu
from jax.experimental.pallas import tpu_sc as plsc

N = 100000
B = 16384
MD = 100
TD = 100
RD = 172
L = 16
NW = 32
BPW = B // NW
RANGE = 3136
NPAD = NW * RANGE

_mesh = plsc.VectorSubcoreMesh(core_axis_name="c", subcore_axis_name="s")


def _wid():
    return lax.axis_index("s") * 2 + lax.axis_index("c")


MP = 128
CH = BPW // 2


@functools.partial(
    pl.kernel,
    mesh=_mesh,
    compiler_params=pltpu.CompilerParams(needs_layout_passes=False),
    out_type=(
        jax.ShapeDtypeStruct((B, MP), jnp.float32),
        jax.ShapeDtypeStruct((B, MP), jnp.float32),
        jax.ShapeDtypeStruct((B,), jnp.float32),
        jax.ShapeDtypeStruct((B,), jnp.float32),
        jax.ShapeDtypeStruct((B,), jnp.int32),
    ),
    scratch_types=[
        pltpu.VMEM((BPW,), jnp.int32),
        pltpu.VMEM((BPW,), jnp.int32),
        pltpu.VMEM((BPW,), jnp.int32),
        pltpu.VMEM((BPW,), jnp.int32),
        pltpu.VMEM((BPW,), jnp.int32),
        pltpu.VMEM((BPW,), jnp.int32),
        pltpu.VMEM((BPW,), jnp.int32),
        pltpu.VMEM((BPW,), jnp.int32),
        pltpu.VMEM((BPW,), jnp.int32),
        pltpu.VMEM((CH, MP), jnp.float32),
        pltpu.VMEM((CH, MP), jnp.float32),
        pltpu.VMEM((BPW,), jnp.int32),
        pltpu.VMEM((BPW,), jnp.float32),
        pltpu.VMEM((BPW,), jnp.float32),
        pltpu.SemaphoreType.DMA,
        pltpu.SemaphoreType.DMA,
        pltpu.SemaphoreType.DMA,
    ],
)
def _sc_gather(mem_hbm, lu_hbm, nid_hbm, dst_hbm, srcd_hbm, ts_hbm, td_hbm,
               h_out, oth_out, trel_out, sel_out, tmax_out,
               nid_v, dst_v, src_v, ts_v, td_v, idx2_v, idxl_v, tev_v, tmx_v,
               hrow_v, orow_v, lu_v, trel_v, sel_v, sem1, sem2, sem3):
    base = _wid() * BPW
    pltpu.sync_copy(nid_hbm.at[pl.ds(base, BPW)], nid_v)
    pltpu.sync_copy(dst_hbm.at[pl.ds(base, BPW)], dst_v)
    pltpu.sync_copy(srcd_hbm.at[pl.ds(base, BPW)], src_v)
    pltpu.sync_copy(ts_hbm.at[pl.ds(base, BPW)], ts_v)
    pltpu.sync_copy(td_hbm.at[pl.ds(base, BPW)], td_v)

    def step(i, carry):
        s = pl.ds(i * L, L)
        ts = ts_v[s]
        td = td_v[s]
        sel = td >= ts
        idx2_v[s] = jnp.where(sel, src_v[s], dst_v[s])
        idxl_v[s] = jnp.where(sel, src_v[s], nid_v[s])
        tev_v[s] = jnp.where(sel, td, ts)
        tmx_v[s] = jnp.maximum(ts, td)
        sel_v[s] = jnp.where(sel, 1.0, 0.0).astype(jnp.float32)
        return carry

    lax.fori_loop(0, BPW // L, step, 0)

    cp3 = pltpu.async_copy(lu_hbm.at[idxl_v], lu_v, sem3)
    for c in range(BPW // CH):
        cp1 = pltpu.async_copy(mem_hbm.at[nid_v.at[pl.ds(c * CH, CH)]], hrow_v, sem1)
        cp2 = pltpu.async_copy(mem_hbm.at[idx2_v.at[pl.ds(c * CH, CH)]], orow_v, sem2)
        cp1.wait()
        cp2.wait()
        pltpu.sync_copy(hrow_v, h_out.at[pl.ds(base + c * CH, CH)])
        pltpu.sync_copy(orow_v, oth_out.at[pl.ds(base + c * CH, CH)])
    cp3.wait()

    def step2(i, carry):
        s = pl.ds(i * L, L)
        trel_v[s] = (tev_v[s] - lu_v[s]).astype(jnp.float32)
        return carry

    lax.fori_loop(0, BPW // L, step2, 0)

    pltpu.sync_copy(trel_v, trel_out.at[pl.ds(base, BPW)])
    pltpu.sync_copy(sel_v, sel_out.at[pl.ds(base, BPW)])
    pltpu.sync_copy(tmx_v, tmax_out.at[pl.ds(base, BPW)])


@functools.partial(
    pl.kernel,
    mesh=_mesh,
    compiler_params=pltpu.CompilerParams(needs_layout_passes=False),
    out_type=jax.ShapeDtypeStruct((NPAD,), jnp.int32),
    scratch_types=[
        pltpu.VMEM((RANGE,), jnp.int32),
        pltpu.VMEM((B,), jnp.int32),
        pltpu.VMEM((B,), jnp.int32),
    ],
)
def _sc_scatter_max(lupad_hbm, nid_hbm, tmax_hbm, luout_hbm, tab_v, nid_v, tmx_v):
    lo = _wid() * RANGE
    pltpu.sync_copy(lupad_hbm.at[pl.ds(lo, RANGE)], tab_v)
    pltpu.sync_copy(nid_hbm, nid_v)
    pltpu.sync_copy(tmax_hbm, tmx_v)

    def step(i, carry):
        s = pl.ds(i * L, L)
        local = nid_v[s] - lo
        tv = tmx_v[s]
        m = (local >= 0) & (local < RANGE)
        cur = plsc.load_gather(tab_v, [local], mask=m)
        act = m & (cur < tv)

        def cond(a):
            return jnp.any(a)

        def body(a):
            plsc.store_scatter(tab_v, [local], tv, mask=a)
            cur2 = plsc.load_gather(tab_v, [local], mask=a)
            return a & (cur2 < tv)

        lax.while_loop(cond, body, act)
        return carry

    lax.fori_loop(0, B // L, step, 0)
    pltpu.sync_copy(tab_v, luout_hbm.at[pl.ds(lo, RANGE)])


@functools.partial(
    pl.kernel,
    mesh=_mesh,
    compiler_params=pltpu.CompilerParams(needs_layout_passes=False),
    out_type=jax.ShapeDtypeStruct((B,), jnp.int32),
    scratch_types=[
        pltpu.VMEM((BPW,), jnp.int32),
        pltpu.VMEM((BPW,), jnp.int32),
        pltpu.SemaphoreType.DMA,
    ],
)
def _sc_last_gather(lut_hbm, nid_hbm, out_hbm, nid_v, val_v, sem):
    base = _wid() * BPW
    pltpu.sync_copy(nid_hbm.at[pl.ds(base, BPW)], nid_v)
    pltpu.async_copy(lut_hbm.at[nid_v], val_v, sem).wait()
    pltpu.sync_copy(val_v, out_hbm.at[pl.ds(base, BPW)])


PADBLK = 2000


def _pad_body(in_ref, out_ref):
    out_ref[...] = jnp.concatenate(
        [in_ref[...], jnp.zeros((PADBLK, MP - MD), jnp.float32)], axis=1)


_pad_mem = pl.pallas_call(
    _pad_body,
    grid=(N // PADBLK,),
    in_specs=[pl.BlockSpec((PADBLK, MD), lambda i: (i, 0))],
    out_specs=pl.BlockSpec((PADBLK, MP), lambda i: (i, 0)),
    out_shape=jax.ShapeDtypeStruct((N, MP), jnp.float32),
)


BLK = 512
G = 384


def _gru_body(sel_ref, trel_ref, h_ref, o_ref, rms_ref, rmd_ref,
              tw_ref, tb_ref, w1_ref, w2_ref, w3_ref, w4_ref,
              whh_ref, bih_ref, bhh_ref, out_ref):
    f32 = jnp.float32
    sel = sel_ref[...]
    h = h_ref[:, 0:MD]
    o = o_ref[:, 0:MD]
    inv = 1.0 - sel
    p1 = sel * o + inv * h
    p2 = sel * h + inv * o
    p3 = sel * rmd_ref[...] + inv * rms_ref[...]
    tenc = jnp.cos(trel_ref[...] * tw_ref[...] + tb_ref[...])
    gi = (jnp.dot(p1, w1_ref[...], preferred_element_type=f32)
          + jnp.dot(p2, w2_ref[...], preferred_element_type=f32)
          + jnp.dot(p3, w3_ref[...], preferred_element_type=f32)
          + jnp.dot(tenc, w4_ref[...], preferred_element_type=f32)
          + bih_ref[...])
    gh = jnp.dot(h, whh_ref[...], preferred_element_type=f32) + bhh_ref[...]
    r = jax.nn.sigmoid(gi[:, 0:128] + gh[:, 0:128])
    z = jax.nn.sigmoid(gi[:, 128:256] + gh[:, 128:256])
    n = jnp.tanh(gi[:, 256:G] + r * gh[:, 256:G])
    out_ref[...] = (1.0 - z[:, 0:MD]) * n[:, 0:MD] + z[:, 0:MD] * h


_gru = pl.pallas_call(
    _gru_body,
    grid=(B // BLK,),
    in_specs=[
        pl.BlockSpec((BLK, 1), lambda i: (i, 0)),
        pl.BlockSpec((BLK, 1), lambda i: (i, 0)),
        pl.BlockSpec((BLK, MP), lambda i: (i, 0)),
        pl.BlockSpec((BLK, MP), lambda i: (i, 0)),
        pl.BlockSpec((BLK, RD), lambda i: (i, 0)),
        pl.BlockSpec((BLK, RD), lambda i: (i, 0)),
        pl.BlockSpec((1, TD), lambda i: (0, 0)),
        pl.BlockSpec((1, TD), lambda i: (0, 0)),
        pl.BlockSpec((MD, G), lambda i: (0, 0)),
        pl.BlockSpec((MD, G), lambda i: (0, 0)),
        pl.BlockSpec((RD, G), lambda i: (0, 0)),
        pl.BlockSpec((TD, G), lambda i: (0, 0)),
        pl.BlockSpec((MD, G), lambda i: (0, 0)),
        pl.BlockSpec((1, G), lambda i: (0, 0)),
        pl.BlockSpec((1, G), lambda i: (0, 0)),
    ],
    out_specs=pl.BlockSpec((BLK, MD), lambda i: (i, 0)),
    out_shape=jax.ShapeDtypeStruct((B, MD), jnp.float32),
)


def _pad_gates(w):
    lead = w.shape[:-1]
    return jnp.pad(w.reshape(lead + (3, MD)),
                   [(0, 0)] * len(lead) + [(0, 0), (0, 28)]).reshape(lead + (G,))


def kernel(memory, last_update, n_id, dst_s, src_d, t_s, t_d,
           raw_msg_s, raw_msg_d, time_w, time_b, W_ih, W_hh, b_ih, b_hh):
    mem_p = _pad_mem(memory)
    h_rows, oth_rows, trel, sel, tmax = _sc_gather(
        mem_p, last_update, n_id, dst_s, src_d, t_s, t_d)

    lu_pad = jnp.pad(last_update, (0, NPAD - N))
    lu_tab = _sc_scatter_max(lu_pad, n_id, tmax)
    new_last = _sc_last_gather(lu_tab, n_id)

    w_iht = _pad_gates(W_ih.T)
    w1 = w_iht[0:MD]
    w2 = w_iht[MD:2 * MD]
    w3 = w_iht[2 * MD:2 * MD + RD]
    w4 = w_iht[2 * MD + RD:]
    whh = _pad_gates(W_hh.T)
    bih = _pad_gates(b_ih).reshape(1, G)
    bhh = _pad_gates(b_hh).reshape(1, G)

    new_mem = _gru(sel.reshape(B, 1), trel.reshape(B, 1), h_rows, oth_rows,
                   raw_msg_s, raw_msg_d,
                   time_w.reshape(1, TD), time_b.reshape(1, TD),
                   w1, w2, w3, w4, whh, bih, bhh)
    import os as _os
    _var = _os.environ.get("SCBAND_VARIANT", "")
    if _var == "nogru":
        new_mem = h_rows[:, :MD] + oth_rows[:, :MD] + trel.reshape(B, 1) + sel.reshape(B, 1)
    if _var == "noscatter":
        new_last = tmax
    return new_mem, new_last


# speedup vs baseline: 11.0217x; 1.8777x over previous
import functools

import jax
import jax.numpy as jnp
from jax import lax
from jax.experimental import pallas as pl
from jax.experimental.pallas import tpu as pltpu
from jax.experimental.pallas import tpu_sc as plsc

N = 100000
B = 16384
MD = 100
TD = 100
RD = 172
L = 16
NW = 32
BPW = B // NW
RANGE = 3136
NPAD = NW * RANGE

_mesh = plsc.VectorSubcoreMesh(core_axis_name="c", subcore_axis_name="s")


def _wid():
    return lax.axis_index("s") * 2 + lax.axis_index("c")


MP = 128
CH = BPW // 2


@functools.partial(
    pl.kernel,
    mesh=_mesh,
    compiler_params=pltpu.CompilerParams(needs_layout_passes=False),
    out_type=(
        jax.ShapeDtypeStruct((B, MP), jnp.float32),
        jax.ShapeDtypeStruct((B, MP), jnp.float32),
        jax.ShapeDtypeStruct((B,), jnp.float32),
        jax.ShapeDtypeStruct((B,), jnp.float32),
        jax.ShapeDtypeStruct((B,), jnp.int32),
    ),
    scratch_types=[
        pltpu.VMEM((BPW,), jnp.int32),
        pltpu.VMEM((BPW,), jnp.int32),
        pltpu.VMEM((BPW,), jnp.int32),
        pltpu.VMEM((BPW,), jnp.int32),
        pltpu.VMEM((BPW,), jnp.int32),
        pltpu.VMEM((BPW,), jnp.int32),
        pltpu.VMEM((BPW,), jnp.int32),
        pltpu.VMEM((BPW,), jnp.int32),
        pltpu.VMEM((BPW,), jnp.int32),
        pltpu.VMEM((CH, MP), jnp.float32),
        pltpu.VMEM((CH, MP), jnp.float32),
        pltpu.VMEM((BPW,), jnp.int32),
        pltpu.VMEM((BPW,), jnp.float32),
        pltpu.VMEM((BPW,), jnp.float32),
        pltpu.SemaphoreType.DMA,
        pltpu.SemaphoreType.DMA,
        pltpu.SemaphoreType.DMA,
    ],
)
def _sc_gather(mem_hbm, lu_hbm, nid_hbm, dst_hbm, srcd_hbm, ts_hbm, td_hbm,
               h_out, oth_out, trel_out, sel_out, tmax_out,
               nid_v, dst_v, src_v, ts_v, td_v, idx2_v, idxl_v, tev_v, tmx_v,
               hrow_v, orow_v, lu_v, trel_v, sel_v, sem1, sem2, sem3):
    base = _wid() * BPW
    pltpu.sync_copy(nid_hbm.at[pl.ds(base, BPW)], nid_v)
    pltpu.sync_copy(dst_hbm.at[pl.ds(base, BPW)], dst_v)
    pltpu.sync_copy(srcd_hbm.at[pl.ds(base, BPW)], src_v)
    pltpu.sync_copy(ts_hbm.at[pl.ds(base, BPW)], ts_v)
    pltpu.sync_copy(td_hbm.at[pl.ds(base, BPW)], td_v)

    def step(i, carry):
        s = pl.ds(i * L, L)
        ts = ts_v[s]
        td = td_v[s]
        sel = td >= ts
        idx2_v[s] = jnp.where(sel, src_v[s], dst_v[s])
        idxl_v[s] = jnp.where(sel, src_v[s], nid_v[s])
        tev_v[s] = jnp.where(sel, td, ts)
        tmx_v[s] = jnp.maximum(ts, td)
        sel_v[s] = jnp.where(sel, 1.0, 0.0).astype(jnp.float32)
        return carry

    lax.fori_loop(0, BPW // L, step, 0)

    cp3 = pltpu.async_copy(lu_hbm.at[idxl_v], lu_v, sem3)
    for c in range(BPW // CH):
        cp1 = pltpu.async_copy(mem_hbm.at[nid_v.at[pl.ds(c * CH, CH)]], hrow_v, sem1)
        cp2 = pltpu.async_copy(mem_hbm.at[idx2_v.at[pl.ds(c * CH, CH)]], orow_v, sem2)
        cp1.wait()
        cp2.wait()
        pltpu.sync_copy(hrow_v, h_out.at[pl.ds(base + c * CH, CH)])
        pltpu.sync_copy(orow_v, oth_out.at[pl.ds(base + c * CH, CH)])
    cp3.wait()

    def step2(i, carry):
        s = pl.ds(i * L, L)
        trel_v[s] = (tev_v[s] - lu_v[s]).astype(jnp.float32)
        return carry

    lax.fori_loop(0, BPW // L, step2, 0)

    pltpu.sync_copy(trel_v, trel_out.at[pl.ds(base, BPW)])
    pltpu.sync_copy(sel_v, sel_out.at[pl.ds(base, BPW)])
    pltpu.sync_copy(tmx_v, tmax_out.at[pl.ds(base, BPW)])


@functools.partial(
    pl.kernel,
    mesh=_mesh,
    compiler_params=pltpu.CompilerParams(needs_layout_passes=False),
    out_type=jax.ShapeDtypeStruct((NPAD,), jnp.int32),
    scratch_types=[
        pltpu.VMEM((RANGE,), jnp.int32),
        pltpu.VMEM((B,), jnp.int32),
        pltpu.VMEM((B,), jnp.int32),
    ],
)
def _sc_scatter_max(lupad_hbm, nid_hbm, tmax_hbm, luout_hbm, tab_v, nid_v, tmx_v):
    lo = _wid() * RANGE
    pltpu.sync_copy(lupad_hbm.at[pl.ds(lo, RANGE)], tab_v)
    pltpu.sync_copy(nid_hbm, nid_v)
    pltpu.sync_copy(tmax_hbm, tmx_v)

    def step(i, carry):
        s = pl.ds(i * L, L)
        local = nid_v[s] - lo
        tv = tmx_v[s]
        m = (local >= 0) & (local < RANGE)
        cur = plsc.load_gather(tab_v, [local], mask=m)
        act = m & (cur < tv)

        def cond(a):
            return jnp.any(a)

        def body(a):
            plsc.store_scatter(tab_v, [local], tv, mask=a)
            cur2 = plsc.load_gather(tab_v, [local], mask=a)
            return a & (cur2 < tv)

        lax.while_loop(cond, body, act)
        return carry

    lax.fori_loop(0, B // L, step, 0)
    pltpu.sync_copy(tab_v, luout_hbm.at[pl.ds(lo, RANGE)])


@functools.partial(
    pl.kernel,
    mesh=_mesh,
    compiler_params=pltpu.CompilerParams(needs_layout_passes=False),
    out_type=jax.ShapeDtypeStruct((B,), jnp.int32),
    scratch_types=[
        pltpu.VMEM((BPW,), jnp.int32),
        pltpu.VMEM((BPW,), jnp.int32),
        pltpu.SemaphoreType.DMA,
    ],
)
def _sc_last_gather(lut_hbm, nid_hbm, out_hbm, nid_v, val_v, sem):
    base = _wid() * BPW
    pltpu.sync_copy(nid_hbm.at[pl.ds(base, BPW)], nid_v)
    pltpu.async_copy(lut_hbm.at[nid_v], val_v, sem).wait()
    pltpu.sync_copy(val_v, out_hbm.at[pl.ds(base, BPW)])


PADBLK = 2000


def _pad_body(in_ref, out_ref):
    out_ref[...] = jnp.concatenate(
        [in_ref[...], jnp.zeros((PADBLK, MP - MD), jnp.float32)], axis=1)


_pad_mem = pl.pallas_call(
    _pad_body,
    grid=(N // PADBLK,),
    in_specs=[pl.BlockSpec((PADBLK, MD), lambda i: (i, 0))],
    out_specs=pl.BlockSpec((PADBLK, MP), lambda i: (i, 0)),
    out_shape=jax.ShapeDtypeStruct((N, MP), jnp.float32),
)


BLK = 512
G = 384


def _gru_body(sel_ref, trel_ref, h_ref, o_ref, rms_ref, rmd_ref,
              tw_ref, tb_ref, w1_ref, w2_ref, w3_ref, w4_ref,
              whh_ref, bih_ref, bhh_ref, out_ref):
    f32 = jnp.float32
    sel = sel_ref[...]
    h = h_ref[:, 0:MD]
    o = o_ref[:, 0:MD]
    inv = 1.0 - sel
    p1 = sel * o + inv * h
    p2 = sel * h + inv * o
    p3 = sel * rmd_ref[...] + inv * rms_ref[...]
    tenc = jnp.cos(trel_ref[...] * tw_ref[...] + tb_ref[...])
    gi = (jnp.dot(p1, w1_ref[...], preferred_element_type=f32)
          + jnp.dot(p2, w2_ref[...], preferred_element_type=f32)
          + jnp.dot(p3, w3_ref[...], preferred_element_type=f32)
          + jnp.dot(tenc, w4_ref[...], preferred_element_type=f32)
          + bih_ref[...])
    gh = jnp.dot(h, whh_ref[...], preferred_element_type=f32) + bhh_ref[...]
    r = jax.nn.sigmoid(gi[:, 0:128] + gh[:, 0:128])
    z = jax.nn.sigmoid(gi[:, 128:256] + gh[:, 128:256])
    n = jnp.tanh(gi[:, 256:G] + r * gh[:, 256:G])
    out_ref[...] = (1.0 - z[:, 0:MD]) * n[:, 0:MD] + z[:, 0:MD] * h


_gru = pl.pallas_call(
    _gru_body,
    grid=(B // BLK,),
    in_specs=[
        pl.BlockSpec((BLK, 1), lambda i: (i, 0)),
        pl.BlockSpec((BLK, 1), lambda i: (i, 0)),
        pl.BlockSpec((BLK, MP), lambda i: (i, 0)),
        pl.BlockSpec((BLK, MP), lambda i: (i, 0)),
        pl.BlockSpec((BLK, RD), lambda i: (i, 0)),
        pl.BlockSpec((BLK, RD), lambda i: (i, 0)),
        pl.BlockSpec((1, TD), lambda i: (0, 0)),
        pl.BlockSpec((1, TD), lambda i: (0, 0)),
        pl.BlockSpec((MD, G), lambda i: (0, 0)),
        pl.BlockSpec((MD, G), lambda i: (0, 0)),
        pl.BlockSpec((RD, G), lambda i: (0, 0)),
        pl.BlockSpec((TD, G), lambda i: (0, 0)),
        pl.BlockSpec((MD, G), lambda i: (0, 0)),
        pl.BlockSpec((1, G), lambda i: (0, 0)),
        pl.BlockSpec((1, G), lambda i: (0, 0)),
    ],
    out_specs=pl.BlockSpec((BLK, MD), lambda i: (i, 0)),
    out_shape=jax.ShapeDtypeStruct((B, MD), jnp.float32),
)


def _pad_gates(w):
    lead = w.shape[:-1]
    return jnp.pad(w.reshape(lead + (3, MD)),
                   [(0, 0)] * len(lead) + [(0, 0), (0, 28)]).reshape(lead + (G,))


def kernel(memory, last_update, n_id, dst_s, src_d, t_s, t_d,
           raw_msg_s, raw_msg_d, time_w, time_b, W_ih, W_hh, b_ih, b_hh):
    mem_p = _pad_mem(memory)
    h_rows, oth_rows, trel, sel, tmax = _sc_gather(
        mem_p, last_update, n_id, dst_s, src_d, t_s, t_d)

    lu_pad = jnp.pad(last_update, (0, NPAD - N))
    lu_tab = _sc_scatter_max(lu_pad, n_id, tmax)
    new_last = _sc_last_gather(lu_tab, n_id)

    w_iht = _pad_gates(W_ih.T)
    w1 = w_iht[0:MD]
    w2 = w_iht[MD:2 * MD]
    w3 = w_iht[2 * MD:2 * MD + RD]
    w4 = w_iht[2 * MD + RD:]
    whh = _pad_gates(W_hh.T)
    bih = _pad_gates(b_ih).reshape(1, G)
    bhh = _pad_gates(b_hh).reshape(1, G)

    new_mem = _gru(sel.reshape(B, 1), trel.reshape(B, 1), h_rows, oth_rows,
                   raw_msg_s, raw_msg_d,
                   time_w.reshape(1, TD), time_b.reshape(1, TD),
                   w1, w2, w3, w4, whh, bih, bhh)
    import os as _os
    _var = _os.environ.get("SCBAND_VARIANT", "")
    if _var == "nogru":
        new_mem = h_rows[:, :MD] + oth_rows[:, :MD] + trel.reshape(B, 1) + sel.reshape(B, 1)
    if _var == "noscatter":
        new_last = tmax
    if _var == "padonly":
        new_mem = mem_p[:B, :MD]
        new_last = last_update[:B]
    return new_mem, new_last
